# baseline (speedup 1.0000x reference)
import functools

import jax
import jax.numpy as jnp
import numpy as np
from jax import lax
from jax.experimental import pallas as pl
from jax.experimental.pallas import tpu as pltpu
from jax.experimental.pallas import tpu_sc as plsc

D = 1024
H = 1024
O = 1024
N = 8
TOK = 2048
TEMP = float(np.e)
NA = 5

_NC = 2
_NS = 16
_NW = _NC * _NS
_TPW = TOK // _NW
_L = 16


def _logits_body(x_ref, wg_ref, bgt_ref, logt_ref, xbf_ref):
    x = x_ref[...]
    logt_ref[...] = jax.lax.dot_general(
        wg_ref[...], x, (((1,), (1,)), ((), ())),
        preferred_element_type=jnp.float32) + bgt_ref[...]
    xbf_ref[...] = x.astype(jnp.bfloat16)


def _sc_gate_body(logt_hbm, wt_hbm, lv, wv):
    wid = lax.axis_index("s") * _NC + lax.axis_index("c")
    base = wid * _TPW
    pltpu.sync_copy(logt_hbm, lv)
    for ch in range(_TPW // _L):
        sc = [lv[e, pl.ds(base + ch * _L, _L)] * (1.0 / TEMP)
              for e in range(N)]
        m = sc[0]
        for e in range(1, N):
            m = jnp.maximum(m, sc[e])
        q = [jnp.exp(s - m) for s in sc]
        ssum = q[0]
        for e in range(1, N):
            ssum = ssum + q[e]
        p = [qe / ssum for qe in q]
        avail = [jnp.full((_L,), 1.0, jnp.float32) for _ in range(N)]
        mask = [jnp.full((_L,), 0.0, jnp.float32) for _ in range(N)]
        for _ in range(NA):
            cand = [p[e] * avail[e] + avail[e] - 1.0 for e in range(N)]
            cm = cand[0]
            for e in range(1, N):
                cm = jnp.maximum(cm, cand[e])
            chosen = jnp.full((_L,), 0.0, jnp.float32)
            for e in range(N):
                ismax = 1.0 + jnp.sign(cand[e] - cm)
                cond = ismax * (1.0 - chosen)
                mask[e] = mask[e] + cond
                avail[e] = avail[e] * (1.0 - cond)
                chosen = chosen + cond
        wsum = mask[0] * p[0]
        for e in range(1, N):
            wsum = wsum + mask[e] * p[e]
        inv = 1.0 / (wsum + 1e-8)
        for e in range(N):
            wv[e, pl.ds(base + ch * _L, _L)] = mask[e] * p[e] * inv
    pltpu.sync_copy(wv.at[:, pl.ds(base, _TPW)],
                    wt_hbm.at[:, pl.ds(base, _TPW)])


def _expert_body(xbf_ref, wt_ref, w1_ref, b1_ref, w2_ref, b2_ref,
                 out_ref, w_ref):
    e = pl.program_id(0)
    h1 = jax.lax.dot_general(
        xbf_ref[...], w1_ref[0].astype(jnp.bfloat16),
        (((1,), (1,)), ((), ())), preferred_element_type=jnp.float32)
    h1 = jnp.maximum(h1 + b1_ref[0], 0.0)
    eo = jax.lax.dot_general(
        h1.astype(jnp.bfloat16), w2_ref[0].astype(jnp.bfloat16),
        (((1,), (1,)), ((), ())), preferred_element_type=jnp.float32)
    eo = eo + b2_ref[0]
    wfull = jnp.transpose(wt_ref[...])

    @pl.when(e == 0)
    def _emit_w():
        w_ref[...] = wfull

    iota = jax.lax.broadcasted_iota(jnp.int32, (TOK, N), 1)
    wcol = jnp.sum(jnp.where(iota == e, wfull, 0.0), axis=1,
                   keepdims=True)
    prev = jnp.where(e == 0, 0.0, out_ref[...])
    out_ref[...] = prev + wcol * eo


@jax.jit
def kernel(x, Wg, bg, W1, b1, W2, b2):
    logt, xbf = pl.pallas_call(
        _logits_body,
        in_specs=[
            pl.BlockSpec((TOK, D), lambda: (0, 0)),
            pl.BlockSpec((N, D), lambda: (0, 0)),
            pl.BlockSpec((N, 1), lambda: (0, 0)),
        ],
        out_specs=[
            pl.BlockSpec((N, TOK), lambda: (0, 0)),
            pl.BlockSpec((TOK, D), lambda: (0, 0)),
        ],
        out_shape=[
            jax.ShapeDtypeStruct((N, TOK), jnp.float32),
            jax.ShapeDtypeStruct((TOK, D), jnp.bfloat16),
        ],
    )(x, Wg, bg.reshape(N, 1))

    sc_gate = functools.partial(
        pl.kernel,
        out_type=jax.ShapeDtypeStruct((N, TOK), jnp.float32),
        mesh=plsc.VectorSubcoreMesh(core_axis_name="c", subcore_axis_name="s"),
        scratch_types=[
            pltpu.VMEM((N, TOK), jnp.float32),
            pltpu.VMEM((N, TOK), jnp.float32),
        ],
    )(_sc_gate_body)
    wt = sc_gate(logt)

    out, w = pl.pallas_call(
        _expert_body,
        grid=(N,),
        in_specs=[
            pl.BlockSpec((TOK, D), lambda e: (0, 0)),
            pl.BlockSpec((N, TOK), lambda e: (0, 0)),
            pl.BlockSpec((1, H, D), lambda e: (e, 0, 0)),
            pl.BlockSpec((1, 1, H), lambda e: (e, 0, 0)),
            pl.BlockSpec((1, O, H), lambda e: (e, 0, 0)),
            pl.BlockSpec((1, 1, O), lambda e: (e, 0, 0)),
        ],
        out_specs=[
            pl.BlockSpec((TOK, O), lambda e: (0, 0)),
            pl.BlockSpec((TOK, N), lambda e: (0, 0)),
        ],
        out_shape=[
            jax.ShapeDtypeStruct((TOK, O), jnp.float32),
            jax.ShapeDtypeStruct((TOK, N), jnp.float32),
        ],
        compiler_params=pltpu.CompilerParams(
            dimension_semantics=("arbitrary",)),
    )(xbf, wt, W1, b1.reshape(N, 1, H), W2, b2.reshape(N, 1, O))
    return (out, w)


# speedup vs baseline: 1.3099x; 1.3099x over previous
import functools

import jax
import jax.numpy as jnp
import numpy as np
from jax.experimental import pallas as pl
from jax.experimental.pallas import tpu as pltpu

D = 1024
H = 1024
O = 1024
N = 8
TOK = 2048
TEMP = float(np.e)
NA = 5


def _moe_body(x_ref, wg_ref, bgt_ref, w1_ref, b1_ref, w2_ref, b2_ref,
              out_ref, w_ref, xbf_ref):
    e = pl.program_id(0)

    @pl.when(e == 0)
    def _gate():
        x = x_ref[...]
        logt = jax.lax.dot_general(
            wg_ref[...], x, (((1,), (1,)), ((), ())),
            preferred_element_type=jnp.float32) + bgt_ref[...]
        s = logt * (1.0 / TEMP)
        m = jnp.max(s, axis=0, keepdims=True)
        q = jnp.exp(s - m)
        p = q / jnp.sum(q, axis=0, keepdims=True)
        iota = jax.lax.broadcasted_iota(jnp.int32, (N, TOK), 0)
        pmk = p
        mask = jnp.zeros_like(p)
        for _ in range(NA):
            cm = jnp.max(pmk, axis=0, keepdims=True)
            first = jnp.min(jnp.where(pmk == cm, iota, N), axis=0,
                            keepdims=True)
            sel = iota == first
            mask = jnp.where(sel, 1.0, mask)
            pmk = jnp.where(sel, -1.0, pmk)
        wm = p * mask
        wt = wm / (jnp.sum(wm, axis=0, keepdims=True) + 1e-8)
        w_ref[...] = jnp.transpose(wt)
        xbf_ref[...] = x.astype(jnp.bfloat16)

    h1 = jax.lax.dot_general(
        xbf_ref[...], w1_ref[0].astype(jnp.bfloat16),
        (((1,), (1,)), ((), ())), preferred_element_type=jnp.float32)
    h1 = jnp.maximum(h1 + b1_ref[0], 0.0)
    eo = jax.lax.dot_general(
        h1.astype(jnp.bfloat16), w2_ref[0].astype(jnp.bfloat16),
        (((1,), (1,)), ((), ())), preferred_element_type=jnp.float32)
    eo = eo + b2_ref[0]
    iota = jax.lax.broadcasted_iota(jnp.int32, (TOK, N), 1)
    wcol = jnp.sum(jnp.where(iota == e, w_ref[...], 0.0), axis=1,
                   keepdims=True)
    prev = jnp.where(e == 0, 0.0, out_ref[...])
    out_ref[...] = prev + wcol * eo


@jax.jit
def kernel(x, Wg, bg, W1, b1, W2, b2):
    out, w = pl.pallas_call(
        _moe_body,
        grid=(N,),
        in_specs=[
            pl.BlockSpec((TOK, D), lambda e: (0, 0)),
            pl.BlockSpec((N, D), lambda e: (0, 0)),
            pl.BlockSpec((N, 1), lambda e: (0, 0)),
            pl.BlockSpec((1, H, D), lambda e: (e, 0, 0)),
            pl.BlockSpec((1, 1, H), lambda e: (e, 0, 0)),
            pl.BlockSpec((1, O, H), lambda e: (e, 0, 0)),
            pl.BlockSpec((1, 1, O), lambda e: (e, 0, 0)),
        ],
        out_specs=[
            pl.BlockSpec((TOK, O), lambda e: (0, 0)),
            pl.BlockSpec((TOK, N), lambda e: (0, 0)),
        ],
        out_shape=[
            jax.ShapeDtypeStruct((TOK, O), jnp.float32),
            jax.ShapeDtypeStruct((TOK, N), jnp.float32),
        ],
        scratch_shapes=[pltpu.VMEM((TOK, D), jnp.bfloat16)],
        compiler_params=pltpu.CompilerParams(
            dimension_semantics=("arbitrary",)),
    )(x, Wg, bg.reshape(N, 1), W1, b1.reshape(N, 1, H), W2,
      b2.reshape(N, 1, O))
    return (out, w)
